# Initial kernel scaffold; baseline (speedup 1.0000x reference)
import jax
import jax.numpy as jnp
from jax.experimental import pallas as pl


def kernel(x, edge_index, n2p_rows, n2p_cols, n2p_vals, e2p_rows, e2p_cols, e2p_vals, pool_rows, pool_cols, pool_vals, atom_W, atom_b, edge_W, edge_b, lrp_W, lrp_b, deg0_W, deg0_b, deg1_W, deg1_b):
    raise NotImplementedError("write your pallas kernel here")


# speedup vs baseline: 10.9929x; 10.9929x over previous
import functools

import jax
import jax.numpy as jnp
from jax import lax
from jax.experimental import pallas as pl
from jax.experimental.pallas import tpu as pltpu
from jax.experimental.pallas import tpu_sc as plsc

N = 50000
E = 800000
L = 16
D = 32
PL = N * L
NLAYERS = 4
NATOM = 28

NC = 2
NS = 16
NW = NC * NS

NPAD = 50176
NPT = 3136
PADN = 51200
EPAD = 819200

_mesh = plsc.VectorSubcoreMesh(core_axis_name="c", subcore_axis_name="s")
f32 = jnp.float32


_DEG_T = 200


@functools.partial(
    pl.kernel,
    mesh=_mesh,
    compiler_params=pltpu.CompilerParams(use_tc_tiling_on_sc=False),
    out_type=jax.ShapeDtypeStruct((NC, NPAD, 16), f32),
    scratch_types=[
        pltpu.VMEM_SHARED((NPAD, 16), f32),
        pltpu.VMEM((_DEG_T, 128), jnp.int32),
        pltpu.VMEM((128, 16), f32),
        pltpu.VMEM((128, 16), f32),
        pltpu.SemaphoreType.DMA,
    ],
)
def _sc_degs(edst_hbm, out_hbm, acc, idxb, onesb, zb, sem):
    cid = lax.axis_index("c")
    t = lax.axis_index("s")

    def fill_ones(r, _):
        onesb[r, pl.ds(0, 16)] = jnp.full((16,), 1.0, f32)
        zb[r, pl.ds(0, 16)] = jnp.zeros((16,), f32)
        return 0

    lax.fori_loop(0, 128, fill_ones, 0)

    def zcp(j, _):
        pltpu.sync_copy(
            zb, acc.at[pl.ds(pl.multiple_of(t * NPT + j * 128, 8), 128)])
        return 0

    lax.fori_loop(0, 24, zcp, 0)
    pltpu.sync_copy(zb.at[pl.ds(0, 64)],
                    acc.at[pl.ds(t * NPT + 24 * 128, 64)])
    plsc.subcore_barrier()

    base = cid * (NS * _DEG_T) + t * _DEG_T
    pltpu.sync_copy(edst_hbm.at[pl.ds(base, _DEG_T)], idxb)

    def scat(j, _):
        pltpu.sync_copy(onesb, acc.at[idxb.at[j]], add=True)
        return 0

    lax.fori_loop(0, _DEG_T, scat, 0)
    plsc.subcore_barrier()
    pltpu.sync_copy(acc.at[pl.ds(t * NPT, NPT)],
                    out_hbm.at[cid, pl.ds(t * NPT, NPT)])


_G_CH = 1024
_G_FULL = 24
_G_TAIL = 424
_G_PW = 25000


@functools.partial(
    pl.kernel,
    mesh=_mesh,
    compiler_params=pltpu.CompilerParams(use_tc_tiling_on_sc=False),
    out_type=jax.ShapeDtypeStruct((PL, D), f32),
    scratch_types=[
        pltpu.VMEM((_G_CH,), jnp.int32),
        pltpu.VMEM((_G_CH, D), f32),
        pltpu.VMEM((_G_TAIL,), jnp.int32),
        pltpu.VMEM((_G_TAIL, D), f32),
        pltpu.SemaphoreType.DMA,
    ],
)
def _sc_gather(h_hbm, cols_hbm, out_hbm, idxb, rowsb, tidxb, trowsb, sem):
    cid = lax.axis_index("c")
    t = lax.axis_index("s")
    base = (t * NC + cid) * _G_PW

    def chunk(j, _):
        off = pl.multiple_of(base + j * _G_CH, 8)
        pltpu.sync_copy(cols_hbm.at[pl.ds(off, _G_CH)], idxb)
        pltpu.async_copy(h_hbm.at[idxb], rowsb, sem).wait()
        pltpu.sync_copy(rowsb, out_hbm.at[pl.ds(off, _G_CH)])
        return 0

    lax.fori_loop(0, _G_FULL, chunk, 0)
    toff = pl.multiple_of(base + _G_FULL * _G_CH, 8)
    pltpu.sync_copy(cols_hbm.at[pl.ds(toff, _G_TAIL)], tidxb)
    pltpu.async_copy(h_hbm.at[tidxb], trowsb, sem).wait()
    pltpu.sync_copy(trowsb, out_hbm.at[pl.ds(toff, _G_TAIL)])


_S_PW = 3200


@functools.partial(
    pl.kernel,
    mesh=_mesh,
    compiler_params=pltpu.CompilerParams(use_tc_tiling_on_sc=False),
    out_type=jax.ShapeDtypeStruct((NPAD, D), f32),
    scratch_types=[
        pltpu.VMEM_SHARED((NPAD, D), f32),
        pltpu.VMEM((128, D), f32),
        pltpu.VMEM((25, 128), jnp.int32),
        pltpu.VMEM((128, D), f32),
        pltpu.VMEM((128, D), f32),
        pltpu.SemaphoreType.DMA,
    ],
)
def _sc_scatter(out2_hbm, prows_hbm, fd_hbm, h_hbm, acc, datab, idxb,
                ab, fb, sem):
    cid = lax.axis_index("c")
    t = lax.axis_index("s")

    @pl.when(cid == 0)
    def _():
        def zrow(r, _):
            datab[r, pl.ds(0, 16)] = jnp.zeros((16,), f32)
            datab[r, pl.ds(16, 16)] = jnp.zeros((16,), f32)
            return 0

        lax.fori_loop(0, 128, zrow, 0)

        def zcp(j, _):
            pltpu.sync_copy(
                datab, acc.at[pl.ds(pl.multiple_of(t * NPT + j * 128, 8),
                                    128)])
            return 0

        lax.fori_loop(0, 24, zcp, 0)
        pltpu.sync_copy(datab.at[pl.ds(0, 64)],
                        acc.at[pl.ds(t * NPT + 24 * 128, 64)])
        plsc.subcore_barrier()

        def ldidx(j, _):
            pltpu.sync_copy(prows_hbm.at[pl.ds(t * _S_PW + j * 128, 128)],
                            idxb.at[j])
            return 0

        lax.fori_loop(0, 25, ldidx, 0)

        def scat(j, _):
            pltpu.sync_copy(
                out2_hbm.at[pl.ds(pl.multiple_of(t * _S_PW + j * 128, 8),
                                  128)], datab)
            pltpu.sync_copy(datab, acc.at[idxb.at[j]], add=True)
            return 0

        lax.fori_loop(0, 25, scat, 0)
        plsc.subcore_barrier()

        base = t * NPT

        def mulrows(nrows):
            def mrow(r, _):
                ab[r, pl.ds(0, 16)] = (ab[r, pl.ds(0, 16)]
                                       * fb[r, pl.ds(0, 16)])
                ab[r, pl.ds(16, 16)] = (ab[r, pl.ds(16, 16)]
                                        * fb[r, pl.ds(16, 16)])
                return 0

            lax.fori_loop(0, nrows, mrow, 0)

        def rb(j, _):
            off = pl.multiple_of(base + j * 128, 8)
            pltpu.sync_copy(acc.at[pl.ds(off, 128)], ab)
            pltpu.sync_copy(fd_hbm.at[pl.ds(off, 128)], fb)
            mulrows(128)
            pltpu.sync_copy(ab, h_hbm.at[pl.ds(off, 128)])
            return 0

        lax.fori_loop(0, 24, rb, 0)
        toff = pl.multiple_of(base + 24 * 128, 8)
        pltpu.sync_copy(acc.at[pl.ds(toff, 64)], ab.at[pl.ds(0, 64)])
        pltpu.sync_copy(fd_hbm.at[pl.ds(toff, 64)], fb.at[pl.ds(0, 64)])
        mulrows(64)
        pltpu.sync_copy(ab.at[pl.ds(0, 64)], h_hbm.at[pl.ds(toff, 64)])


_BN = 1024
_NBLK = 50


def _mm_body(g_ref, v_ref, e_ref, p_ref, W2_ref, evW_ref, b_ref, EXP_ref,
             o_ref):
    i = pl.program_id(0)
    ve = jnp.dot(v_ref[...], EXP_ref[...], preferred_element_type=f32)
    acc = jnp.dot(g_ref[...] * ve, W2_ref[...], preferred_element_type=f32)
    acc = acc + jnp.dot(e_ref[...], evW_ref[...], preferred_element_type=f32)
    acc = jnp.maximum(acc + b_ref[...], 0.0) * p_ref[...]
    rows = i * _BN + lax.broadcasted_iota(jnp.int32, (_BN, D), 0)
    o_ref[...] = jnp.where(rows < N, acc, 0.0)


def _tc_mm(g2, valsNL, e2pNL, pv2d, W2i, evWi, bi, EXP):
    clamp = lambda i: (jnp.minimum(i, _NBLK - 2), 0)
    return pl.pallas_call(
        _mm_body,
        grid=(_NBLK,),
        in_specs=[
            pl.BlockSpec((_BN, L * D), clamp),
            pl.BlockSpec((_BN, L), clamp),
            pl.BlockSpec((_BN, L), clamp),
            pl.BlockSpec((_BN, 1), clamp),
            pl.BlockSpec((L * D, D), lambda i: (0, 0)),
            pl.BlockSpec((L, D), lambda i: (0, 0)),
            pl.BlockSpec((1, D), lambda i: (0, 0)),
            pl.BlockSpec((L, L * D), lambda i: (0, 0)),
        ],
        out_specs=pl.BlockSpec((_BN, D), lambda i: (i, 0)),
        out_shape=jax.ShapeDtypeStruct((PADN, D), f32),
    )(g2, valsNL, e2pNL, pv2d, W2i, evWi, bi, EXP)


_PBLK = 49


def _prep_body(x_ref, dp_ref, aWT_ref, ab_ref, d0w_ref, d0b_ref, d1WT_ref,
               d1b_ref, h0_ref, f0_ref, f1_ref, f2_ref, f3_ref):
    h0 = jnp.dot(x_ref[...], aWT_ref[...], preferred_element_type=f32)
    h0_ref[...] = h0 + ab_ref[...]
    degs = dp_ref[0, :, 0:1] + dp_ref[1, :, 0:1]
    outs = (f0_ref, f1_ref, f2_ref, f3_ref)
    for i in range(NLAYERS):
        tmp = jnp.maximum(degs * d0w_ref[i:i + 1, :] + d0b_ref[i:i + 1, :],
                          0.0)
        outs[i][...] = (jnp.dot(tmp, d1WT_ref[i],
                                preferred_element_type=f32)
                        + d1b_ref[i:i + 1, :])


def _tc_prep(x, dparts, atom_WT, atom_b2, deg0w, deg0_b, deg1_WT, deg1_b):
    fshape = jax.ShapeDtypeStruct((NPAD, D), f32)
    return pl.pallas_call(
        _prep_body,
        grid=(_PBLK,),
        in_specs=[
            pl.BlockSpec((_BN, NATOM), lambda i: (i, 0)),
            pl.BlockSpec((NC, _BN, 16), lambda i: (0, i, 0)),
            pl.BlockSpec((NATOM, D), lambda i: (0, 0)),
            pl.BlockSpec((1, D), lambda i: (0, 0)),
            pl.BlockSpec((NLAYERS, 2 * D), lambda i: (0, 0)),
            pl.BlockSpec((NLAYERS, 2 * D), lambda i: (0, 0)),
            pl.BlockSpec((NLAYERS, 2 * D, D), lambda i: (0, 0, 0)),
            pl.BlockSpec((NLAYERS, D), lambda i: (0, 0)),
        ],
        out_specs=[pl.BlockSpec((_BN, D), lambda i: (i, 0))] * 5,
        out_shape=[fshape] * 5,
    )(x, dparts, atom_WT, atom_b2, deg0w, deg0_b, deg1_WT, deg1_b)


def kernel(x, edge_index, n2p_rows, n2p_cols, n2p_vals, e2p_rows, e2p_cols,
           e2p_vals, pool_rows, pool_cols, pool_vals, atom_W, atom_b,
           edge_W, edge_b, lrp_W, lrp_b, deg0_W, deg0_b, deg1_W, deg1_b):
    valsNL = n2p_vals.reshape(N, L)
    e2pNL = e2p_vals.reshape(N, L)
    pv2d = pool_vals.reshape(N, 1)
    prows_p = jnp.concatenate(
        [pool_rows, jnp.full((PADN - N,), N, jnp.int32)])
    edst2d = jnp.concatenate(
        [edge_index[1], jnp.full((EPAD - E,), N, jnp.int32)]).reshape(
            EPAD // 128, 128)
    ev = edge_W[:, 0] + edge_b
    W2 = jnp.transpose(lrp_W, (0, 3, 1, 2)).reshape(NLAYERS, L * D, D)
    evW = jnp.einsum('b,nbcl->nlc', ev, lrp_W)
    EXP = jnp.kron(jnp.eye(L, dtype=f32), jnp.ones((1, D), f32))
    atom_WT = atom_W.T
    atom_b2 = atom_b.reshape(1, D)
    deg0w = deg0_W[:, :, 0]
    deg1_WT = jnp.transpose(deg1_W, (0, 2, 1))

    dparts = _sc_degs(edst2d)
    h, f0, f1, f2, f3 = _tc_prep(x, dparts, atom_WT, atom_b2, deg0w,
                                 deg0_b, deg1_WT, deg1_b)
    fds = (f0, f1, f2, f3)
    for i in range(NLAYERS):
        g = _sc_gather(h, n2p_cols)
        out2 = _tc_mm(g.reshape(N, L * D), valsNL, e2pNL, pv2d, W2[i],
                      evW[i], lrp_b[i], EXP)
        h = _sc_scatter(out2, prows_p, fds[i])
    return h[:N]
